# Initial kernel scaffold; baseline (speedup 1.0000x reference)
import jax
import jax.numpy as jnp
from jax.experimental import pallas as pl


def kernel(x, edge_index, batch, conv_w, conv_b, fc1_w, fc1_b, rel_w, rel_b, root_w, bn_g, bn_b, fc2_w, fc2_b):
    raise NotImplementedError("write your pallas kernel here")


# speedup vs baseline: 1.3388x; 1.3388x over previous
import jax
import jax.numpy as jnp
from jax import lax
from jax.experimental import pallas as pl
from jax.experimental.pallas import tpu as pltpu
from jax.experimental.pallas import tpu_sc as plsc

N = 50000
E = 800000
G = 64
DH = 128
L = 100
D = 112

NC = 2
NS = 16
NT = NC * NS
NPAD = 50176
GB = 128
EB2 = 1024
NB2 = 816
EPAD2 = NB2 * EB2
TPG = EPAD2 // NT // GB
WIN = 1032

RB = 1024
NB = NPAD // RB


def _sc_gather(xpad, ssp):
    def body(xpad_hbm, src_hbm, hr_hbm, gsrc, rows, sem):
        ci = lax.axis_index("c")
        si = lax.axis_index("s")
        eoff = (ci * NS + si) * (TPG * GB)

        def g2(g, c):
            off = pl.multiple_of(eoff + g * GB, GB)
            pltpu.sync_copy(src_hbm.at[pl.ds(off, GB)], gsrc)
            pltpu.async_copy(xpad_hbm.at[gsrc], rows, sem).wait()
            pltpu.sync_copy(rows, hr_hbm.at[pl.ds(off, GB)])
            return c

        lax.fori_loop(0, TPG, g2, 0)

    mesh = plsc.VectorSubcoreMesh(core_axis_name="c", subcore_axis_name="s")
    return pl.kernel(
        body,
        out_type=jax.ShapeDtypeStruct((EPAD2, D), jnp.float32),
        mesh=mesh,
        compiler_params=pltpu.CompilerParams(use_tc_tiling_on_sc=False),
        scratch_types=[
            pltpu.VMEM((GB,), jnp.int32),
            pltpu.VMEM((GB, D), jnp.float32),
            pltpu.SemaphoreType.DMA,
        ])(xpad, ssp)


def _tc_segsum(hr, dst3, bases):
    def body(bases_ref, rows_ref, d3_ref, xa_ref):
        i = pl.program_id(0)

        @pl.when(i == 0)
        def _():
            xa_ref[...] = jnp.zeros_like(xa_ref)

        base = bases_ref[i]
        rel = d3_ref[0, 0, :] - base
        mask = (lax.broadcasted_iota(jnp.int32, (WIN, EB2), 0)
                == rel[None, :]).astype(jnp.float32)
        xa_ref[pl.ds(base, WIN), :] += jnp.dot(
            mask, rows_ref[...], preferred_element_type=jnp.float32)

    return pl.pallas_call(
        body,
        grid=(NB2,),
        in_specs=[
            pl.BlockSpec(memory_space=pltpu.SMEM),
            pl.BlockSpec((EB2, D), lambda i: (i, 0)),
            pl.BlockSpec((1, 1, EB2), lambda i: (i, 0, 0)),
        ],
        out_specs=pl.BlockSpec((NPAD, D), lambda i: (0, 0)),
        out_shape=jax.ShapeDtypeStruct((NPAD, D), jnp.float32),
    )(bases, hr, dst3)


def _tc_head(xa, xp, batch3, wrp, wop, bng2, bnb2, fc2_w, fc2_b2):
    def body(xa_ref, xp_ref, b3_ref, wr_ref, wo_ref, bng_ref, bnb_ref,
             f2w_ref, f2b_ref, y_ref, s1_ref, s2_ref, cnt_ref):
        i = pl.program_id(0)

        @pl.when(i == 0)
        def _():
            s1_ref[...] = jnp.zeros_like(s1_ref)
            s2_ref[...] = jnp.zeros_like(s2_ref)
            cnt_ref[...] = jnp.zeros_like(cnt_ref)
            y_ref[...] = jnp.zeros_like(y_ref)

        h2 = (jnp.dot(xa_ref[...], wr_ref[...],
                      preferred_element_type=jnp.float32)
              + jnp.dot(xp_ref[...], wo_ref[...],
                        preferred_element_type=jnp.float32))
        b = b3_ref[0, 0, :]
        mask = (lax.broadcasted_iota(jnp.int32, (G, RB), 0)
                == b[None, :]).astype(jnp.float32)
        s1_ref[...] += jnp.dot(mask, h2, preferred_element_type=jnp.float32)
        s2_ref[...] += jnp.dot(mask, h2 * h2,
                               preferred_element_type=jnp.float32)
        cnt_ref[...] += jnp.broadcast_to(
            jnp.sum(mask, axis=1, keepdims=True), (G, DH))

        @pl.when(i == NB - 1)
        def _():
            s1 = s1_ref[...]
            s2 = s2_ref[...]
            cnt = cnt_ref[...]
            t1 = jnp.sum(s1, axis=0)
            t2 = jnp.sum(s2, axis=0)
            mean = t1 / N
            var = t2 / N - mean * mean
            a = bng_ref[0, :] / jnp.sqrt(var + 1e-5)
            c = bnb_ref[0, :] - mean * a
            cc = jnp.maximum(cnt, 1.0)
            pooled = ((a * a)[None, :] * s2 / cc
                      + (2.0 * a * c)[None, :] * s1 / cc
                      + (c * c)[None, :])
            hh = jnp.log(jnp.maximum(pooled, 1e-6))
            y = lax.dot_general(hh, f2w_ref[...], (((1,), (1,)), ((), ())),
                                preferred_element_type=jnp.float32)
            y_ref[...] = jax.nn.sigmoid(y + f2b_ref[...])

    out_shape = (
        jax.ShapeDtypeStruct((G, 3), jnp.float32),
        jax.ShapeDtypeStruct((G, DH), jnp.float32),
        jax.ShapeDtypeStruct((G, DH), jnp.float32),
        jax.ShapeDtypeStruct((G, DH), jnp.float32),
    )
    in_specs = [
        pl.BlockSpec((RB, D), lambda i: (i, 0)),
        pl.BlockSpec((RB, D), lambda i: (i, 0)),
        pl.BlockSpec((1, 1, RB), lambda i: (i, 0, 0)),
        pl.BlockSpec((D, DH), lambda i: (0, 0)),
        pl.BlockSpec((D, DH), lambda i: (0, 0)),
        pl.BlockSpec((1, DH), lambda i: (0, 0)),
        pl.BlockSpec((1, DH), lambda i: (0, 0)),
        pl.BlockSpec((3, DH), lambda i: (0, 0)),
        pl.BlockSpec((G, 3), lambda i: (0, 0)),
    ]
    out_specs = (
        pl.BlockSpec((G, 3), lambda i: (0, 0)),
        pl.BlockSpec((G, DH), lambda i: (0, 0)),
        pl.BlockSpec((G, DH), lambda i: (0, 0)),
        pl.BlockSpec((G, DH), lambda i: (0, 0)),
    )
    y, _, _, _ = pl.pallas_call(
        body, grid=(NB,), in_specs=in_specs, out_specs=out_specs,
        out_shape=out_shape)(
            xa, xp, batch3, wrp, wop, bng2, bnb2, fc2_w, fc2_b2)
    return y


def kernel(x, edge_index, batch, conv_w, conv_b, fc1_w, fc1_b, rel_w, rel_b,
           root_w, bn_g, bn_b, fc2_w, fc2_b):
    f = fc1_w[0]
    K = conv_w.shape[2]
    P = K // 2
    shifts = []
    for k in range(K):
        s = P - k
        if s >= 0:
            gk = jnp.concatenate([f[s:], jnp.zeros((s,), f.dtype)])
        else:
            gk = jnp.concatenate([jnp.zeros((-s,), f.dtype), f[:s]])
        shifts.append(gk)
    shifts = jnp.stack(shifts)
    A = jnp.einsum("ck,kl->cl", conv_w[:, 0, :], shifts)
    const_c = conv_b * jnp.sum(f) + fc1_b[0]

    wr = A.T @ rel_w.T
    cr = rel_w @ const_c
    wo = A.T @ root_w.T
    bo = root_w @ const_c
    wrp = jnp.zeros((D, DH), jnp.float32).at[:L].set(wr).at[L].set(cr)
    wop = jnp.zeros((D, DH), jnp.float32).at[:L].set(wo).at[L].set(bo + rel_b)

    xp = (jnp.zeros((NPAD, D), jnp.float32)
          .at[:N, :L].set(x).at[:N, L].set(1.0))

    order = jnp.argsort(edge_index[1])
    ss = edge_index[0][order]
    ds = edge_index[1][order]
    ssp = jnp.concatenate([ss, jnp.zeros((EPAD2 - E,), jnp.int32)])
    dsp = jnp.concatenate([ds, jnp.full((EPAD2 - E,), NPAD + 4096,
                                        jnp.int32)])
    dst3 = dsp.reshape(NB2, 1, EB2)
    bases = jnp.minimum((dst3[:, 0, 0] // 8) * 8, NPAD - WIN)

    batch3 = jnp.concatenate(
        [batch, jnp.full((NPAD - N,), G, jnp.int32)]).reshape(NB, 1, RB)
    bng2 = bn_g.reshape(1, DH)
    bnb2 = bn_b.reshape(1, DH)
    fc2_b2 = jnp.broadcast_to(fc2_b[None, :], (G, 3))

    hr = _sc_gather(xp, ssp)
    xa = _tc_segsum(hr, dst3, bases)
    return _tc_head(xa, xp, batch3, wrp, wop, bng2, bnb2, fc2_w, fc2_b2)
